# baseline (speedup 1.0000x reference)
import functools

import jax
import jax.numpy as jnp
from jax import lax
from jax.experimental import pallas as pl
from jax.experimental.pallas import tpu as pltpu
from jax.experimental.pallas import tpu_sc as plsc

NSC = 2
NTILE = 16
NW = NSC * NTILE
LANES = 16
D_SM = 64

CA = 448
CB = 448


def _round_up(x, m):
    return (x + m - 1) // m * m


def _dense_body(n_real, br, hp_ref, wp_ref, bp_ref, wl_ref, bl_ref, wr_ref,
                br_ref, l_ref, r_ref):
    i = pl.program_id(0)
    h = jnp.tanh(
        jnp.dot(hp_ref[...], wp_ref[...], preferred_element_type=jnp.float32)
        + bp_ref[...])
    lv = jnp.tanh(
        jnp.dot(h, wl_ref[...], preferred_element_type=jnp.float32)
        + bl_ref[...])
    rv = jnp.tanh(
        jnp.dot(h, wr_ref[...], preferred_element_type=jnp.float32)
        + br_ref[...])
    rid = i * br + lax.broadcasted_iota(jnp.int32, (br, D_SM), 0)
    keep = rid < n_real
    l_ref[...] = jnp.where(keep, lv, 0.0).astype(jnp.bfloat16)
    r_ref[...] = jnp.where(keep, rv, 0.0).astype(jnp.bfloat16)


def _dense_tables(hp, W_proj, b_proj, W_left, b_left, W_right, b_right, n_real,
                  npad):
    br = 1024 if npad % 1024 == 0 else (512 if npad % 512 == 0 else 256)
    grid = (npad // br,)
    full = lambda s: pl.BlockSpec(s, lambda i: (0, 0))
    return pl.pallas_call(
        functools.partial(_dense_body, n_real, br),
        grid=grid,
        in_specs=[
            pl.BlockSpec((br, hp.shape[1]), lambda i: (i, 0)),
            full(W_proj.shape),
            full((1, D_SM)),
            full(W_left.shape),
            full((1, D_SM)),
            full(W_right.shape),
            full((1, D_SM)),
        ],
        out_specs=[
            pl.BlockSpec((br, D_SM), lambda i: (i, 0)),
            pl.BlockSpec((br, D_SM), lambda i: (i, 0)),
        ],
        out_shape=[
            jax.ShapeDtypeStruct((npad, D_SM), jnp.bfloat16),
            jax.ShapeDtypeStruct((npad, D_SM), jnp.bfloat16),
        ],
    )(hp, W_proj, b_proj.reshape(1, -1), W_left, b_left.reshape(1, -1),
      W_right, b_right.reshape(1, -1))


def _pass_a_body(npad, epad, l_hbm, r_hbm, ij_hbm, w_hbm, d_hbm,
                 vi0, vj0, lr0, rr0, w0, vi1, vj1, lr1, rr1, w1,
                 zbuf, denom_sp,
                 isem0, lsem0, rsem0, wsem0, isem1, lsem1, rsem1, wsem1):
    cid = lax.axis_index("c")
    sid = lax.axis_index("s")
    wid = cid * NTILE + sid
    npts = npad // NTILE

    def _z(i, _):
        zbuf[pl.ds(i * LANES, LANES)] = jnp.zeros((LANES,), jnp.float32)
        return 0
    lax.fori_loop(0, npts // LANES, _z, 0)
    pltpu.sync_copy(zbuf, denom_sp.at[pl.ds(sid * npts, npts)])
    plsc.subcore_barrier()

    ept = epad // NW
    base_e = wid * ept
    nch = ept // CA

    bufs = [
        dict(vi=vi0, vj=vj0, lr=lr0, rr=rr0, w=w0, isem=isem0, lsem=lsem0,
             rsem=rsem0, wsem=wsem0),
        dict(vi=vi1, vj=vj1, lr=lr1, rr=rr1, w=w1, isem=isem1, lsem=lsem1,
             rsem=rsem1, wsem=wsem1),
    ]

    def issue_idx(k, b):
        off = base_e + k * CA
        pltpu.async_copy(ij_hbm.at[pl.ds(off, CA)], b["vi"], b["isem"])
        pltpu.async_copy(ij_hbm.at[pl.ds(epad + off, CA)], b["vj"], b["isem"])

    def wait_idx(k, b):
        off = base_e + k * CA
        pltpu.make_async_copy(ij_hbm.at[pl.ds(off, CA)], b["vi"],
                              b["isem"]).wait()
        pltpu.make_async_copy(ij_hbm.at[pl.ds(epad + off, CA)], b["vj"],
                              b["isem"]).wait()

    def issue_rows(b):
        pltpu.async_copy(l_hbm.at[b["vi"]], b["lr"], b["lsem"])
        pltpu.async_copy(r_hbm.at[b["vj"]], b["rr"], b["rsem"])

    def wait_rows(b):
        pltpu.make_async_copy(l_hbm.at[b["vi"]], b["lr"], b["lsem"]).wait()
        pltpu.make_async_copy(r_hbm.at[b["vj"]], b["rr"], b["rsem"]).wait()

    issue_idx(0, bufs[0])
    wait_idx(0, bufs[0])
    issue_rows(bufs[0])
    issue_idx(1, bufs[1])

    def halfstep(k, cur, nxt):
        @pl.when(k + 1 < nch)
        def _():
            wait_idx(k + 1, nxt)
            issue_rows(nxt)

        wait_rows(cur)
        lrows, rrows, w_loc = cur["lr"], cur["rr"], cur["w"]

        lane = lax.iota(jnp.int32, LANES)

        def _grp(g, _):
            eb = g * LANES
            out = jnp.zeros((LANES,), jnp.float32)
            for i in range(LANES):
                e = eb + i
                s = jnp.zeros((LANES,), jnp.float32)
                for kk in range(D_SM // (2 * LANES)):
                    lv = lrows[e, pl.ds(kk * 2 * LANES, 2 * LANES)]
                    rv = rrows[e, pl.ds(kk * 2 * LANES, 2 * LANES)]
                    l0, l1 = plsc.unpack(lv, format=plsc.PackFormat.INTERLEAVED)
                    r0, r1 = plsc.unpack(rv, format=plsc.PackFormat.INTERLEAVED)
                    s = s + l0 * r0 + l1 * r1
                for sh in (8, 4, 2, 1):
                    s = s + jnp.take(s, lane ^ sh)
                out = jnp.where(lane == i, s, out)
            w_loc[pl.ds(eb, LANES)] = jnp.exp(out)
            return 0
        lax.fori_loop(0, CA // LANES, _grp, 0, unroll=1)

        off = base_e + k * CA
        wd = pltpu.async_copy(w_loc, w_hbm.at[pl.ds(off, CA)], cur["wsem"])
        pltpu.sync_copy(w_loc, denom_sp.at[cur["vi"]], add=True)
        wd.wait()

        @pl.when(k + 2 < nch)
        def _():
            issue_idx(k + 2, cur)

    def _iter(k2, _):
        halfstep(2 * k2, bufs[0], bufs[1])
        halfstep(2 * k2 + 1, bufs[1], bufs[0])
        return 0
    lax.fori_loop(0, nch // 2, _iter, 0)

    plsc.subcore_barrier()
    pltpu.sync_copy(denom_sp.at[pl.ds(sid * npts, npts)],
                    d_hbm.at[pl.ds(cid * npad + sid * npts, npts)])


def _pass_a(L, R, ij_p, npad, epad):
    mesh = plsc.VectorSubcoreMesh(core_axis_name="c", subcore_axis_name="s",
                                  num_cores=NSC, num_subcores=NTILE)
    return pl.kernel(
        functools.partial(_pass_a_body, npad, epad),
        out_type=[
            jax.ShapeDtypeStruct((epad,), jnp.float32),
            jax.ShapeDtypeStruct((NSC * npad,), jnp.float32),
        ],
        mesh=mesh,
        compiler_params=pltpu.CompilerParams(needs_layout_passes=False, use_tc_tiling_on_sc=False),
        scratch_types=(
            [pltpu.VMEM((CA,), jnp.int32),
             pltpu.VMEM((CA,), jnp.int32),
             pltpu.VMEM((CA, D_SM), jnp.bfloat16),
             pltpu.VMEM((CA, D_SM), jnp.bfloat16),
             pltpu.VMEM((CA,), jnp.float32)] * 2
            + [pltpu.VMEM((npad // NTILE,), jnp.float32),
               pltpu.VMEM_SHARED((npad,), jnp.float32)]
            + [pltpu.SemaphoreType.DMA] * 8
        ),
    )(L, R, ij_p)


def _pass_b_body(npad, epad, att_hbm, d_hbm, ij_hbm, w_hbm, o_hbm,
                 d0b, ab, qb,
                 vi0, vj0, w0, qv0, ea0, vi1, vj1, w1, qv1, ea1,
                 q_sp, out_sp,
                 isem0, qsem0, isem1, qsem1):
    cid = lax.axis_index("c")
    sid = lax.axis_index("s")
    wid = cid * NTILE + sid
    npts = npad // NTILE
    nsl = pl.ds(sid * npts, npts)

    pltpu.sync_copy(d_hbm.at[pl.ds(sid * npts, npts)], d0b)
    pltpu.sync_copy(d_hbm.at[pl.ds(npad + sid * npts, npts)], qb)
    pltpu.sync_copy(att_hbm.at[pl.ds(sid * npts, npts)], ab)

    def _q(i, _):
        ds_ = pl.ds(i * LANES, LANES)
        den = jnp.maximum(d0b[ds_] + qb[ds_], 1e-30)
        qb[ds_] = ab[ds_] / den
        return 0
    lax.fori_loop(0, npts // LANES, _q, 0)
    pltpu.sync_copy(qb, q_sp.at[nsl])

    def _z(i, _):
        qb[pl.ds(i * LANES, LANES)] = jnp.zeros((LANES,), jnp.float32)
        return 0
    lax.fori_loop(0, npts // LANES, _z, 0)
    pltpu.sync_copy(qb, out_sp.at[nsl])
    plsc.subcore_barrier()

    ept = epad // NW
    base_e = wid * ept
    nch = ept // CB

    bufs = [
        dict(vi=vi0, vj=vj0, w=w0, qv=qv0, ea=ea0, isem=isem0, qsem=qsem0),
        dict(vi=vi1, vj=vj1, w=w1, qv=qv1, ea=ea1, isem=isem1, qsem=qsem1),
    ]

    def issue_idx(k, b):
        off = base_e + k * CB
        pltpu.async_copy(ij_hbm.at[pl.ds(off, CB)], b["vi"], b["isem"])
        pltpu.async_copy(ij_hbm.at[pl.ds(epad + off, CB)], b["vj"], b["isem"])
        pltpu.async_copy(w_hbm.at[pl.ds(off, CB)], b["w"], b["isem"])

    def wait_idx(k, b):
        off = base_e + k * CB
        pltpu.make_async_copy(ij_hbm.at[pl.ds(off, CB)], b["vi"],
                              b["isem"]).wait()
        pltpu.make_async_copy(ij_hbm.at[pl.ds(epad + off, CB)], b["vj"],
                              b["isem"]).wait()
        pltpu.make_async_copy(w_hbm.at[pl.ds(off, CB)], b["w"],
                              b["isem"]).wait()

    def issue_q(b):
        pltpu.async_copy(q_sp.at[b["vi"]], b["qv"], b["qsem"])

    def wait_q(b):
        pltpu.make_async_copy(q_sp.at[b["vi"]], b["qv"], b["qsem"]).wait()

    issue_idx(0, bufs[0])
    wait_idx(0, bufs[0])
    issue_q(bufs[0])
    issue_idx(1, bufs[1])

    def halfstep(k, cur, nxt):
        @pl.when(k + 1 < nch)
        def _():
            wait_idx(k + 1, nxt)
            issue_q(nxt)

        wait_q(cur)
        w_loc, qv, ea = cur["w"], cur["qv"], cur["ea"]

        def _e(g, _):
            ds_ = pl.ds(g * LANES, LANES)
            ea[ds_] = w_loc[ds_] * qv[ds_]
            return 0
        lax.fori_loop(0, CB // LANES, _e, 0)
        pltpu.sync_copy(ea, out_sp.at[cur["vj"]], add=True)

        @pl.when(k + 2 < nch)
        def _():
            issue_idx(k + 2, cur)

    def _iter(k2, _):
        halfstep(2 * k2, bufs[0], bufs[1])
        halfstep(2 * k2 + 1, bufs[1], bufs[0])
        return 0
    lax.fori_loop(0, nch // 2, _iter, 0)

    plsc.subcore_barrier()
    pltpu.sync_copy(out_sp.at[nsl],
                    o_hbm.at[pl.ds(cid * npad + sid * npts, npts)])


def _pass_b(att_p, d, ij_p, w, npad, epad):
    mesh = plsc.VectorSubcoreMesh(core_axis_name="c", subcore_axis_name="s",
                                  num_cores=NSC, num_subcores=NTILE)
    return pl.kernel(
        functools.partial(_pass_b_body, npad, epad),
        out_type=jax.ShapeDtypeStruct((NSC * npad,), jnp.float32),
        mesh=mesh,
        compiler_params=pltpu.CompilerParams(needs_layout_passes=False, use_tc_tiling_on_sc=False),
        scratch_types=(
            [pltpu.VMEM((npad // NTILE,), jnp.float32)] * 3
            + [pltpu.VMEM((CB,), jnp.int32),
               pltpu.VMEM((CB,), jnp.int32),
               pltpu.VMEM((CB,), jnp.float32),
               pltpu.VMEM((CB,), jnp.float32),
               pltpu.VMEM((CB,), jnp.float32)] * 2
            + [pltpu.VMEM_SHARED((npad,), jnp.float32),
               pltpu.VMEM_SHARED((npad,), jnp.float32)]
            + [pltpu.SemaphoreType.DMA] * 4
        ),
    )(att_p, d, ij_p, w)


def _final_body(n_real, o_ref, out_ref):
    a = o_ref[0] + o_ref[1]
    rid = (lax.broadcasted_iota(jnp.int32, a.shape, 0) * 128 +
           lax.broadcasted_iota(jnp.int32, a.shape, 1))
    a = jnp.where(rid < n_real, a, 0.0)
    tot = jnp.sum(a)
    out_ref[...] = a / jnp.maximum(tot, 1e-20)


def _finalize(o, npad, n_real):
    rows = npad // 128
    o3 = o.reshape(NSC, rows, 128)
    return pl.pallas_call(
        functools.partial(_final_body, n_real),
        in_specs=[pl.BlockSpec((NSC, rows, 128), lambda: (0, 0, 0))],
        out_specs=pl.BlockSpec((rows, 128), lambda: (0, 0)),
        out_shape=jax.ShapeDtypeStruct((rows, 128), jnp.float32),
    )(o3)


def kernel(node_attention, hidden, selected_edges, W_proj, b_proj, W_left,
           b_left, W_right, b_right):
    n = hidden.shape[1]
    e = selected_edges.shape[0]
    npad = _round_up(n, LANES * NTILE)
    epad = _round_up(e, NW * CA * 2)

    hp = hidden[0]
    att_p = jnp.pad(node_attention[0], (0, npad - n))
    junk = n + jnp.arange(epad - e, dtype=jnp.int32) % (npad - n)
    ij = selected_edges[:, 1:3].T
    ij_p = jnp.concatenate(
        [ij, jnp.broadcast_to(junk, (2, epad - e))], axis=1).reshape(-1)

    L, R = _dense_tables(hp, W_proj, b_proj, W_left, b_left, W_right,
                         b_right, n, npad)
    w, d = _pass_a(L, R, ij_p, npad, epad)
    o = _pass_b(att_p, d, ij_p, w, npad, epad)
    out = _finalize(o, npad, n)
    return out.reshape(1, npad)[:, :n]


# speedup vs baseline: 1.0723x; 1.0723x over previous
import functools

import jax
import jax.numpy as jnp
from jax import lax
from jax.experimental import pallas as pl
from jax.experimental.pallas import tpu as pltpu
from jax.experimental.pallas import tpu_sc as plsc

NSC = 2
NTILE = 16
NW = NSC * NTILE
LANES = 16
D_SM = 64

CA = 224
CB = 448
NBUF = 4


def _round_up(x, m):
    return (x + m - 1) // m * m


def _dense_body(n_real, br, hp_ref, wp_ref, bp_ref, wl_ref, bl_ref, wr_ref,
                br_ref, l_ref, r_ref):
    i = pl.program_id(0)
    h = jnp.tanh(
        jnp.dot(hp_ref[...], wp_ref[...], preferred_element_type=jnp.float32)
        + bp_ref[...])
    lv = jnp.tanh(
        jnp.dot(h, wl_ref[...], preferred_element_type=jnp.float32)
        + bl_ref[...])
    rv = jnp.tanh(
        jnp.dot(h, wr_ref[...], preferred_element_type=jnp.float32)
        + br_ref[...])
    rid = i * br + lax.broadcasted_iota(jnp.int32, (br, D_SM), 0)
    keep = rid < n_real
    l_ref[...] = jnp.where(keep, lv, 0.0).astype(jnp.bfloat16)
    r_ref[...] = jnp.where(keep, rv, 0.0).astype(jnp.bfloat16)


def _dense_tables(hp, W_proj, b_proj, W_left, b_left, W_right, b_right, n_real,
                  npad):
    br = 1024 if npad % 1024 == 0 else (512 if npad % 512 == 0 else 256)
    grid = (npad // br,)
    full = lambda s: pl.BlockSpec(s, lambda i: (0, 0))
    return pl.pallas_call(
        functools.partial(_dense_body, n_real, br),
        grid=grid,
        in_specs=[
            pl.BlockSpec((br, hp.shape[1]), lambda i: (i, 0)),
            full(W_proj.shape),
            full((1, D_SM)),
            full(W_left.shape),
            full((1, D_SM)),
            full(W_right.shape),
            full((1, D_SM)),
        ],
        out_specs=[
            pl.BlockSpec((br, D_SM), lambda i: (i, 0)),
            pl.BlockSpec((br, D_SM), lambda i: (i, 0)),
        ],
        out_shape=[
            jax.ShapeDtypeStruct((npad, D_SM), jnp.bfloat16),
            jax.ShapeDtypeStruct((npad, D_SM), jnp.bfloat16),
        ],
    )(hp, W_proj, b_proj.reshape(1, -1), W_left, b_left.reshape(1, -1),
      W_right, b_right.reshape(1, -1))


def _pass_a_body(npad, epad, l_hbm, r_hbm, vi_hbm, vj_hbm, w_hbm, d_hbm,
                 vi0, vj0, lr0, rr0, w0, vi1, vj1, lr1, rr1, w1,
                 vi2, vj2, lr2, rr2, w2, vi3, vj3, lr3, rr3, w3,
                 zbuf, denom_sp,
                 isem0, lsem0, rsem0, wsem0, isem1, lsem1, rsem1, wsem1,
                 isem2, lsem2, rsem2, wsem2, isem3, lsem3, rsem3, wsem3):
    cid = lax.axis_index("c")
    sid = lax.axis_index("s")
    wid = cid * NTILE + sid
    npts = npad // NTILE

    def _z(i, _):
        zbuf[pl.ds(i * LANES, LANES)] = jnp.zeros((LANES,), jnp.float32)
        return 0
    lax.fori_loop(0, npts // LANES, _z, 0)
    pltpu.sync_copy(zbuf, denom_sp.at[pl.ds(sid * npts, npts)])
    plsc.subcore_barrier()

    ept = epad // NW
    base_e = wid * ept
    nch = ept // CA

    bufs = [
        dict(vi=vi0, vj=vj0, lr=lr0, rr=rr0, w=w0, isem=isem0, lsem=lsem0,
             rsem=rsem0, wsem=wsem0),
        dict(vi=vi1, vj=vj1, lr=lr1, rr=rr1, w=w1, isem=isem1, lsem=lsem1,
             rsem=rsem1, wsem=wsem1),
        dict(vi=vi2, vj=vj2, lr=lr2, rr=rr2, w=w2, isem=isem2, lsem=lsem2,
             rsem=rsem2, wsem=wsem2),
        dict(vi=vi3, vj=vj3, lr=lr3, rr=rr3, w=w3, isem=isem3, lsem=lsem3,
             rsem=rsem3, wsem=wsem3),
    ]

    def issue_idx(k, b):
        off = base_e + k * CA
        pltpu.async_copy(vi_hbm.at[pl.ds(off, CA)], b["vi"], b["isem"])
        pltpu.async_copy(vj_hbm.at[pl.ds(off, CA)], b["vj"], b["isem"])

    def wait_idx(k, b):
        off = base_e + k * CA
        pltpu.make_async_copy(vi_hbm.at[pl.ds(off, CA)], b["vi"],
                              b["isem"]).wait()
        pltpu.make_async_copy(vj_hbm.at[pl.ds(off, CA)], b["vj"],
                              b["isem"]).wait()

    def issue_rows(b):
        pltpu.async_copy(l_hbm.at[b["vi"]], b["lr"], b["lsem"])
        pltpu.async_copy(r_hbm.at[b["vj"]], b["rr"], b["rsem"])

    def wait_rows(b):
        pltpu.make_async_copy(l_hbm.at[b["vi"]], b["lr"], b["lsem"]).wait()
        pltpu.make_async_copy(r_hbm.at[b["vj"]], b["rr"], b["rsem"]).wait()

    issue_idx(0, bufs[0])
    issue_idx(1, bufs[1])
    wait_idx(0, bufs[0])
    issue_rows(bufs[0])
    wait_idx(1, bufs[1])
    issue_rows(bufs[1])
    issue_idx(2, bufs[2])
    issue_idx(3, bufs[3])

    lane = lax.iota(jnp.int32, LANES)

    def halfstep(k, cur, two):
        @pl.when(k + 2 < nch)
        def _():
            wait_idx(k + 2, two)
            issue_rows(two)

        wait_rows(cur)
        lrows, rrows, w_loc = cur["lr"], cur["rr"], cur["w"]

        def _grp(g, _):
            eb = g * LANES
            out = jnp.zeros((LANES,), jnp.float32)
            for i in range(LANES):
                e = eb + i
                s = jnp.zeros((LANES,), jnp.float32)
                for kk in range(D_SM // (2 * LANES)):
                    lv = lrows[e, pl.ds(kk * 2 * LANES, 2 * LANES)]
                    rv = rrows[e, pl.ds(kk * 2 * LANES, 2 * LANES)]
                    l0, l1 = plsc.unpack(lv, format=plsc.PackFormat.INTERLEAVED)
                    r0, r1 = plsc.unpack(rv, format=plsc.PackFormat.INTERLEAVED)
                    s = s + l0 * r0 + l1 * r1
                for sh in (8, 4, 2, 1):
                    s = s + jnp.take(s, lane ^ sh)
                out = jnp.where(lane == i, s, out)
            w_loc[pl.ds(eb, LANES)] = jnp.exp(out)
            return 0
        lax.fori_loop(0, CA // LANES, _grp, 0, unroll=1)

        off = base_e + k * CA
        wd = pltpu.async_copy(w_loc, w_hbm.at[pl.ds(off, CA)], cur["wsem"])
        pltpu.sync_copy(w_loc, denom_sp.at[cur["vi"]], add=True)
        wd.wait()

        @pl.when(k + NBUF < nch)
        def _():
            issue_idx(k + NBUF, cur)

    def _iter(k4, _):
        for j in range(NBUF):
            halfstep(NBUF * k4 + j, bufs[j], bufs[(j + 2) % NBUF])
        return 0
    lax.fori_loop(0, nch // NBUF, _iter, 0)

    plsc.subcore_barrier()
    pltpu.sync_copy(denom_sp.at[pl.ds(sid * npts, npts)],
                    d_hbm.at[pl.ds(cid * npad + sid * npts, npts)])


def _pass_a(L, R, vi_p, vj_p, npad, epad):
    mesh = plsc.VectorSubcoreMesh(core_axis_name="c", subcore_axis_name="s",
                                  num_cores=NSC, num_subcores=NTILE)
    return pl.kernel(
        functools.partial(_pass_a_body, npad, epad),
        out_type=[
            jax.ShapeDtypeStruct((epad,), jnp.float32),
            jax.ShapeDtypeStruct((NSC * npad,), jnp.float32),
        ],
        mesh=mesh,
        compiler_params=pltpu.CompilerParams(needs_layout_passes=False, use_tc_tiling_on_sc=False),
        scratch_types=(
            [pltpu.VMEM((CA,), jnp.int32),
             pltpu.VMEM((CA,), jnp.int32),
             pltpu.VMEM((CA, D_SM), jnp.bfloat16),
             pltpu.VMEM((CA, D_SM), jnp.bfloat16),
             pltpu.VMEM((CA,), jnp.float32)] * 4
            + [pltpu.VMEM((npad // NTILE,), jnp.float32),
               pltpu.VMEM_SHARED((npad,), jnp.float32)]
            + [pltpu.SemaphoreType.DMA] * 16
        ),
    )(L, R, vi_p, vj_p)


def _pass_b_body(npad, epad, att_hbm, d_hbm, vi_hbm, vj_hbm, w_hbm, o_hbm,
                 d0b, ab, qb,
                 vi0, vj0, w0, qv0, ea0, vi1, vj1, w1, qv1, ea1,
                 q_sp, out_sp,
                 isem0, qsem0, isem1, qsem1):
    cid = lax.axis_index("c")
    sid = lax.axis_index("s")
    wid = cid * NTILE + sid
    npts = npad // NTILE
    nsl = pl.ds(sid * npts, npts)

    pltpu.sync_copy(d_hbm.at[pl.ds(sid * npts, npts)], d0b)
    pltpu.sync_copy(d_hbm.at[pl.ds(npad + sid * npts, npts)], qb)
    pltpu.sync_copy(att_hbm.at[pl.ds(sid * npts, npts)], ab)

    def _q(i, _):
        ds_ = pl.ds(i * LANES, LANES)
        den = jnp.maximum(d0b[ds_] + qb[ds_], 1e-30)
        qb[ds_] = ab[ds_] / den
        return 0
    lax.fori_loop(0, npts // LANES, _q, 0)
    pltpu.sync_copy(qb, q_sp.at[nsl])

    def _z(i, _):
        qb[pl.ds(i * LANES, LANES)] = jnp.zeros((LANES,), jnp.float32)
        return 0
    lax.fori_loop(0, npts // LANES, _z, 0)
    pltpu.sync_copy(qb, out_sp.at[nsl])
    plsc.subcore_barrier()

    ept = epad // NW
    base_e = wid * ept
    nch = ept // CB

    bufs = [
        dict(vi=vi0, vj=vj0, w=w0, qv=qv0, ea=ea0, isem=isem0, qsem=qsem0),
        dict(vi=vi1, vj=vj1, w=w1, qv=qv1, ea=ea1, isem=isem1, qsem=qsem1),
    ]

    def issue_idx(k, b):
        off = base_e + k * CB
        pltpu.async_copy(vi_hbm.at[pl.ds(off, CB)], b["vi"], b["isem"])
        pltpu.async_copy(vj_hbm.at[pl.ds(off, CB)], b["vj"], b["isem"])
        pltpu.async_copy(w_hbm.at[pl.ds(off, CB)], b["w"], b["isem"])

    def wait_idx(k, b):
        off = base_e + k * CB
        pltpu.make_async_copy(vi_hbm.at[pl.ds(off, CB)], b["vi"],
                              b["isem"]).wait()
        pltpu.make_async_copy(vj_hbm.at[pl.ds(off, CB)], b["vj"],
                              b["isem"]).wait()
        pltpu.make_async_copy(w_hbm.at[pl.ds(off, CB)], b["w"],
                              b["isem"]).wait()

    def issue_q(b):
        pltpu.async_copy(q_sp.at[b["vi"]], b["qv"], b["qsem"])

    def wait_q(b):
        pltpu.make_async_copy(q_sp.at[b["vi"]], b["qv"], b["qsem"]).wait()

    issue_idx(0, bufs[0])
    wait_idx(0, bufs[0])
    issue_q(bufs[0])
    issue_idx(1, bufs[1])

    def halfstep(k, cur, nxt):
        @pl.when(k + 1 < nch)
        def _():
            wait_idx(k + 1, nxt)
            issue_q(nxt)

        wait_q(cur)
        w_loc, qv, ea = cur["w"], cur["qv"], cur["ea"]

        def _e(g, _):
            ds_ = pl.ds(g * LANES, LANES)
            ea[ds_] = w_loc[ds_] * qv[ds_]
            return 0
        lax.fori_loop(0, CB // LANES, _e, 0)
        pltpu.sync_copy(ea, out_sp.at[cur["vj"]], add=True)

        @pl.when(k + 2 < nch)
        def _():
            issue_idx(k + 2, cur)

    def _iter(k2, _):
        halfstep(2 * k2, bufs[0], bufs[1])
        halfstep(2 * k2 + 1, bufs[1], bufs[0])
        return 0
    lax.fori_loop(0, nch // 2, _iter, 0)

    plsc.subcore_barrier()
    pltpu.sync_copy(out_sp.at[nsl],
                    o_hbm.at[pl.ds(cid * npad + sid * npts, npts)])


def _pass_b(att_p, d, vi_p, vj_p, w, npad, epad):
    mesh = plsc.VectorSubcoreMesh(core_axis_name="c", subcore_axis_name="s",
                                  num_cores=NSC, num_subcores=NTILE)
    return pl.kernel(
        functools.partial(_pass_b_body, npad, epad),
        out_type=jax.ShapeDtypeStruct((NSC * npad,), jnp.float32),
        mesh=mesh,
        compiler_params=pltpu.CompilerParams(needs_layout_passes=False, use_tc_tiling_on_sc=False),
        scratch_types=(
            [pltpu.VMEM((npad // NTILE,), jnp.float32)] * 3
            + [pltpu.VMEM((CB,), jnp.int32),
               pltpu.VMEM((CB,), jnp.int32),
               pltpu.VMEM((CB,), jnp.float32),
               pltpu.VMEM((CB,), jnp.float32),
               pltpu.VMEM((CB,), jnp.float32)] * 2
            + [pltpu.VMEM_SHARED((npad,), jnp.float32),
               pltpu.VMEM_SHARED((npad,), jnp.float32)]
            + [pltpu.SemaphoreType.DMA] * 4
        ),
    )(att_p, d, vi_p, vj_p, w)


def _final_body(n_real, o_ref, out_ref):
    a = o_ref[0] + o_ref[1]
    rid = (lax.broadcasted_iota(jnp.int32, a.shape, 0) * 128 +
           lax.broadcasted_iota(jnp.int32, a.shape, 1))
    a = jnp.where(rid < n_real, a, 0.0)
    tot = jnp.sum(a)
    out_ref[...] = a / jnp.maximum(tot, 1e-20)


def _finalize(o, npad, n_real):
    rows = npad // 128
    o3 = o.reshape(NSC, rows, 128)
    return pl.pallas_call(
        functools.partial(_final_body, n_real),
        in_specs=[pl.BlockSpec((NSC, rows, 128), lambda: (0, 0, 0))],
        out_specs=pl.BlockSpec((rows, 128), lambda: (0, 0)),
        out_shape=jax.ShapeDtypeStruct((rows, 128), jnp.float32),
    )(o3)


def kernel(node_attention, hidden, selected_edges, W_proj, b_proj, W_left,
           b_left, W_right, b_right):
    n = hidden.shape[1]
    e = selected_edges.shape[0]
    npad = _round_up(n, LANES * NTILE)
    epad = _round_up(e, NW * CA * NBUF)

    hp = hidden[0]
    att_p = jnp.pad(node_attention[0], (0, npad - n))
    junk = n + jnp.arange(epad - e, dtype=jnp.int32) % (npad - n)
    ij = selected_edges[:, 1:3].T
    ij_p = jnp.concatenate(
        [ij, jnp.broadcast_to(junk, (2, epad - e))], axis=1)
    vi_p = ij_p[0]
    vj_p = ij_p[1]

    L, R = _dense_tables(hp, W_proj, b_proj, W_left, b_left, W_right,
                         b_right, n, npad)
    w, d = _pass_a(L, R, vi_p, vj_p, npad, epad)
    o = _pass_b(att_p, d, vi_p, vj_p, w, npad, epad)
    out = _finalize(o, npad, n)
    return out.reshape(1, npad)[:, :n]


# speedup vs baseline: 1.1063x; 1.0317x over previous
import functools

import jax
import jax.numpy as jnp
from jax import lax
from jax.experimental import pallas as pl
from jax.experimental.pallas import tpu as pltpu
from jax.experimental.pallas import tpu_sc as plsc

NSC = 2
NTILE = 16
NW = NSC * NTILE
LANES = 16
D_SM = 64

CA = 224
CB = 448
NBUF = 4


def _round_up(x, m):
    return (x + m - 1) // m * m


def _dense_body(n_real, br, hp_ref, wp_ref, bp_ref, wl_ref, bl_ref, wr_ref,
                br_ref, l_ref, r_ref):
    i = pl.program_id(0)
    h = jnp.tanh(
        jnp.dot(hp_ref[...], wp_ref[...], preferred_element_type=jnp.float32)
        + bp_ref[...])
    lv = jnp.tanh(
        jnp.dot(h, wl_ref[...], preferred_element_type=jnp.float32)
        + bl_ref[...])
    rv = jnp.tanh(
        jnp.dot(h, wr_ref[...], preferred_element_type=jnp.float32)
        + br_ref[...])
    rid = i * br + lax.broadcasted_iota(jnp.int32, (br, D_SM), 0)
    keep = rid < n_real
    l_ref[...] = jnp.where(keep, lv, 0.0).astype(jnp.bfloat16)
    r_ref[...] = jnp.where(keep, rv, 0.0).astype(jnp.bfloat16)


def _dense_tables(hp, W_proj, b_proj, W_left, b_left, W_right, b_right, n_real,
                  npad):
    br = 1024 if npad % 1024 == 0 else (512 if npad % 512 == 0 else 256)
    grid = (npad // br,)
    full = lambda s: pl.BlockSpec(s, lambda i: (0, 0))
    return pl.pallas_call(
        functools.partial(_dense_body, n_real, br),
        grid=grid,
        in_specs=[
            pl.BlockSpec((br, hp.shape[1]), lambda i: (i, 0)),
            full(W_proj.shape),
            full((1, D_SM)),
            full(W_left.shape),
            full((1, D_SM)),
            full(W_right.shape),
            full((1, D_SM)),
        ],
        out_specs=[
            pl.BlockSpec((br, D_SM), lambda i: (i, 0)),
            pl.BlockSpec((br, D_SM), lambda i: (i, 0)),
        ],
        out_shape=[
            jax.ShapeDtypeStruct((npad, D_SM), jnp.bfloat16),
            jax.ShapeDtypeStruct((npad, D_SM), jnp.bfloat16),
        ],
    )(hp, W_proj, b_proj.reshape(1, -1), W_left, b_left.reshape(1, -1),
      W_right, b_right.reshape(1, -1))


def _pass_a_body(npad, epad, l_hbm, r_hbm, vi_hbm, vj_hbm, w_hbm, d_hbm,
                 vi0, vj0, lr0, rr0, w0, vi1, vj1, lr1, rr1, w1,
                 vi2, vj2, lr2, rr2, w2, vi3, vj3, lr3, rr3, w3,
                 zbuf, denom_sp,
                 isem0, lsem0, rsem0, wsem0, isem1, lsem1, rsem1, wsem1,
                 isem2, lsem2, rsem2, wsem2, isem3, lsem3, rsem3, wsem3):
    cid = lax.axis_index("c")
    sid = lax.axis_index("s")
    wid = cid * NTILE + sid
    npts = npad // NTILE

    def _z(i, _):
        zbuf[pl.ds(i * LANES, LANES)] = jnp.zeros((LANES,), jnp.float32)
        return 0
    lax.fori_loop(0, npts // LANES, _z, 0)
    pltpu.sync_copy(zbuf, denom_sp.at[pl.ds(sid * npts, npts)])
    plsc.subcore_barrier()

    ept = epad // NW
    base_e = wid * ept
    nch = ept // CA

    bufs = [
        dict(vi=vi0, vj=vj0, lr=lr0, rr=rr0, w=w0, isem=isem0, lsem=lsem0,
             rsem=rsem0, wsem=wsem0),
        dict(vi=vi1, vj=vj1, lr=lr1, rr=rr1, w=w1, isem=isem1, lsem=lsem1,
             rsem=rsem1, wsem=wsem1),
        dict(vi=vi2, vj=vj2, lr=lr2, rr=rr2, w=w2, isem=isem2, lsem=lsem2,
             rsem=rsem2, wsem=wsem2),
        dict(vi=vi3, vj=vj3, lr=lr3, rr=rr3, w=w3, isem=isem3, lsem=lsem3,
             rsem=rsem3, wsem=wsem3),
    ]

    def issue_idx(k, b):
        off = base_e + k * CA
        pltpu.async_copy(vi_hbm.at[pl.ds(off, CA)], b["vi"], b["isem"])
        pltpu.async_copy(vj_hbm.at[pl.ds(off, CA)], b["vj"], b["isem"])

    def wait_idx(k, b):
        off = base_e + k * CA
        pltpu.make_async_copy(vi_hbm.at[pl.ds(off, CA)], b["vi"],
                              b["isem"]).wait()
        pltpu.make_async_copy(vj_hbm.at[pl.ds(off, CA)], b["vj"],
                              b["isem"]).wait()

    def issue_rows(b):
        pltpu.async_copy(l_hbm.at[b["vi"]], b["lr"], b["lsem"])
        pltpu.async_copy(r_hbm.at[b["vj"]], b["rr"], b["rsem"])

    def wait_rows(b):
        pltpu.make_async_copy(l_hbm.at[b["vi"]], b["lr"], b["lsem"]).wait()
        pltpu.make_async_copy(r_hbm.at[b["vj"]], b["rr"], b["rsem"]).wait()

    issue_idx(0, bufs[0])
    issue_idx(1, bufs[1])
    wait_idx(0, bufs[0])
    issue_rows(bufs[0])
    wait_idx(1, bufs[1])
    issue_rows(bufs[1])
    issue_idx(2, bufs[2])
    issue_idx(3, bufs[3])

    lane = lax.iota(jnp.int32, LANES)

    def halfstep(k, cur, two):
        @pl.when(k + 2 < nch)
        def _():
            wait_idx(k + 2, two)
            issue_rows(two)

        wait_rows(cur)
        lrows, rrows, w_loc = cur["lr"], cur["rr"], cur["w"]

        @pl.when(k >= NBUF)
        def _():
            old = base_e + (k - NBUF) * CA
            pltpu.make_async_copy(w_loc, w_hbm.at[pl.ds(old, CA)],
                                  cur["wsem"]).wait()

        def _grp(g, _):
            eb = g * LANES
            out = jnp.zeros((LANES,), jnp.float32)
            for i in range(LANES):
                e = eb + i
                s = jnp.zeros((LANES,), jnp.float32)
                for kk in range(D_SM // (2 * LANES)):
                    lv = lrows[e, pl.ds(kk * 2 * LANES, 2 * LANES)]
                    rv = rrows[e, pl.ds(kk * 2 * LANES, 2 * LANES)]
                    l0, l1 = plsc.unpack(lv, format=plsc.PackFormat.INTERLEAVED)
                    r0, r1 = plsc.unpack(rv, format=plsc.PackFormat.INTERLEAVED)
                    s = s + l0 * r0 + l1 * r1
                for sh in (8, 4, 2, 1):
                    s = s + jnp.take(s, lane ^ sh)
                out = jnp.where(lane == i, s, out)
            w_loc[pl.ds(eb, LANES)] = jnp.exp(out)
            return 0
        lax.fori_loop(0, CA // LANES, _grp, 0, unroll=1)

        off = base_e + k * CA
        pltpu.sync_copy(w_loc, denom_sp.at[cur["vi"]], add=True)
        pltpu.async_copy(w_loc, w_hbm.at[pl.ds(off, CA)], cur["wsem"])

        @pl.when(k + NBUF < nch)
        def _():
            issue_idx(k + NBUF, cur)

    def _iter(k4, _):
        for j in range(NBUF):
            halfstep(NBUF * k4 + j, bufs[j], bufs[(j + 2) % NBUF])
        return 0
    lax.fori_loop(0, nch // NBUF, _iter, 0)
    for j in range(NBUF):
        old = base_e + (nch - NBUF + j) * CA
        pltpu.make_async_copy(bufs[j]["w"], w_hbm.at[pl.ds(old, CA)],
                              bufs[j]["wsem"]).wait()

    plsc.subcore_barrier()
    pltpu.sync_copy(denom_sp.at[pl.ds(sid * npts, npts)],
                    d_hbm.at[pl.ds(cid * npad + sid * npts, npts)])


def _pass_a(L, R, vi_p, vj_p, npad, epad):
    mesh = plsc.VectorSubcoreMesh(core_axis_name="c", subcore_axis_name="s",
                                  num_cores=NSC, num_subcores=NTILE)
    return pl.kernel(
        functools.partial(_pass_a_body, npad, epad),
        out_type=[
            jax.ShapeDtypeStruct((epad,), jnp.float32),
            jax.ShapeDtypeStruct((NSC * npad,), jnp.float32),
        ],
        mesh=mesh,
        compiler_params=pltpu.CompilerParams(needs_layout_passes=False, use_tc_tiling_on_sc=False),
        scratch_types=(
            [pltpu.VMEM((CA,), jnp.int32),
             pltpu.VMEM((CA,), jnp.int32),
             pltpu.VMEM((CA, D_SM), jnp.bfloat16),
             pltpu.VMEM((CA, D_SM), jnp.bfloat16),
             pltpu.VMEM((CA,), jnp.float32)] * 4
            + [pltpu.VMEM((npad // NTILE,), jnp.float32),
               pltpu.VMEM_SHARED((npad,), jnp.float32)]
            + [pltpu.SemaphoreType.DMA] * 16
        ),
    )(L, R, vi_p, vj_p)


def _pass_b_body(npad, epad, att_hbm, d_hbm, vi_hbm, vj_hbm, w_hbm, o_hbm,
                 d0b, ab, qb,
                 vi0, vj0, w0, qv0, ea0, vjs0, vi1, vj1, w1, qv1, ea1, vjs1,
                 q_sp, out_sp,
                 isem0, qsem0, ssem0, isem1, qsem1, ssem1):
    cid = lax.axis_index("c")
    sid = lax.axis_index("s")
    wid = cid * NTILE + sid
    npts = npad // NTILE
    nsl = pl.ds(sid * npts, npts)

    pltpu.sync_copy(d_hbm.at[pl.ds(sid * npts, npts)], d0b)
    pltpu.sync_copy(d_hbm.at[pl.ds(npad + sid * npts, npts)], qb)
    pltpu.sync_copy(att_hbm.at[pl.ds(sid * npts, npts)], ab)

    def _q(i, _):
        ds_ = pl.ds(i * LANES, LANES)
        den = jnp.maximum(d0b[ds_] + qb[ds_], 1e-30)
        qb[ds_] = ab[ds_] / den
        return 0
    lax.fori_loop(0, npts // LANES, _q, 0)
    pltpu.sync_copy(qb, q_sp.at[nsl])

    def _z(i, _):
        qb[pl.ds(i * LANES, LANES)] = jnp.zeros((LANES,), jnp.float32)
        return 0
    lax.fori_loop(0, npts // LANES, _z, 0)
    pltpu.sync_copy(qb, out_sp.at[nsl])
    plsc.subcore_barrier()

    ept = epad // NW
    base_e = wid * ept
    nch = ept // CB

    bufs = [
        dict(vi=vi0, vj=vj0, w=w0, qv=qv0, ea=ea0, vjs=vjs0, isem=isem0,
             qsem=qsem0, ssem=ssem0),
        dict(vi=vi1, vj=vj1, w=w1, qv=qv1, ea=ea1, vjs=vjs1, isem=isem1,
             qsem=qsem1, ssem=ssem1),
    ]

    def issue_idx(k, b):
        off = base_e + k * CB
        pltpu.async_copy(vi_hbm.at[pl.ds(off, CB)], b["vi"], b["isem"])
        pltpu.async_copy(vj_hbm.at[pl.ds(off, CB)], b["vj"], b["isem"])
        pltpu.async_copy(w_hbm.at[pl.ds(off, CB)], b["w"], b["isem"])

    def wait_idx(k, b):
        off = base_e + k * CB
        pltpu.make_async_copy(vi_hbm.at[pl.ds(off, CB)], b["vi"],
                              b["isem"]).wait()
        pltpu.make_async_copy(vj_hbm.at[pl.ds(off, CB)], b["vj"],
                              b["isem"]).wait()
        pltpu.make_async_copy(w_hbm.at[pl.ds(off, CB)], b["w"],
                              b["isem"]).wait()

    def issue_q(b):
        pltpu.async_copy(q_sp.at[b["vi"]], b["qv"], b["qsem"])

    def wait_q(b):
        pltpu.make_async_copy(q_sp.at[b["vi"]], b["qv"], b["qsem"]).wait()

    issue_idx(0, bufs[0])
    wait_idx(0, bufs[0])
    issue_q(bufs[0])
    issue_idx(1, bufs[1])

    def halfstep(k, cur, nxt):
        @pl.when(k + 1 < nch)
        def _():
            wait_idx(k + 1, nxt)
            issue_q(nxt)

        wait_q(cur)
        w_loc, qv, ea, vjs = cur["w"], cur["qv"], cur["ea"], cur["vjs"]

        @pl.when(k >= 2)
        def _():
            pltpu.make_async_copy(ea, out_sp.at[vjs], cur["ssem"]).wait()

        def _e(g, _):
            ds_ = pl.ds(g * LANES, LANES)
            ea[ds_] = w_loc[ds_] * qv[ds_]
            vjs[ds_] = cur["vj"][ds_]
            return 0
        lax.fori_loop(0, CB // LANES, _e, 0)
        pltpu.async_copy(ea, out_sp.at[vjs], cur["ssem"], add=True)

        @pl.when(k + 2 < nch)
        def _():
            issue_idx(k + 2, cur)

    def _iter(k2, _):
        halfstep(2 * k2, bufs[0], bufs[1])
        halfstep(2 * k2 + 1, bufs[1], bufs[0])
        return 0
    lax.fori_loop(0, nch // 2, _iter, 0)
    for j in range(2):
        pltpu.make_async_copy(bufs[j]["ea"], out_sp.at[bufs[j]["vjs"]],
                              bufs[j]["ssem"]).wait()

    plsc.subcore_barrier()
    pltpu.sync_copy(out_sp.at[nsl],
                    o_hbm.at[pl.ds(cid * npad + sid * npts, npts)])


def _pass_b(att_p, d, vi_p, vj_p, w, npad, epad):
    mesh = plsc.VectorSubcoreMesh(core_axis_name="c", subcore_axis_name="s",
                                  num_cores=NSC, num_subcores=NTILE)
    return pl.kernel(
        functools.partial(_pass_b_body, npad, epad),
        out_type=jax.ShapeDtypeStruct((NSC * npad,), jnp.float32),
        mesh=mesh,
        compiler_params=pltpu.CompilerParams(needs_layout_passes=False, use_tc_tiling_on_sc=False),
        scratch_types=(
            [pltpu.VMEM((npad // NTILE,), jnp.float32)] * 3
            + [pltpu.VMEM((CB,), jnp.int32),
               pltpu.VMEM((CB,), jnp.int32),
               pltpu.VMEM((CB,), jnp.float32),
               pltpu.VMEM((CB,), jnp.float32),
               pltpu.VMEM((CB,), jnp.float32),
               pltpu.VMEM((CB,), jnp.int32)] * 2
            + [pltpu.VMEM_SHARED((npad,), jnp.float32),
               pltpu.VMEM_SHARED((npad,), jnp.float32)]
            + [pltpu.SemaphoreType.DMA] * 6
        ),
    )(att_p, d, vi_p, vj_p, w)


def _final_body(n_real, o_ref, out_ref):
    a = o_ref[0] + o_ref[1]
    rid = (lax.broadcasted_iota(jnp.int32, a.shape, 0) * 128 +
           lax.broadcasted_iota(jnp.int32, a.shape, 1))
    a = jnp.where(rid < n_real, a, 0.0)
    tot = jnp.sum(a)
    out_ref[...] = a / jnp.maximum(tot, 1e-20)


def _finalize(o, npad, n_real):
    rows = npad // 128
    o3 = o.reshape(NSC, rows, 128)
    return pl.pallas_call(
        functools.partial(_final_body, n_real),
        in_specs=[pl.BlockSpec((NSC, rows, 128), lambda: (0, 0, 0))],
        out_specs=pl.BlockSpec((rows, 128), lambda: (0, 0)),
        out_shape=jax.ShapeDtypeStruct((rows, 128), jnp.float32),
    )(o3)


def kernel(node_attention, hidden, selected_edges, W_proj, b_proj, W_left,
           b_left, W_right, b_right):
    n = hidden.shape[1]
    e = selected_edges.shape[0]
    npad = _round_up(n, LANES * NTILE)
    epad = _round_up(e, NW * CA * NBUF)

    hp = hidden[0]
    att_p = jnp.pad(node_attention[0], (0, npad - n))
    junk = n + jnp.arange(epad - e, dtype=jnp.int32) % (npad - n)
    ij = selected_edges[:, 1:3].T
    ij_p = jnp.concatenate(
        [ij, jnp.broadcast_to(junk, (2, epad - e))], axis=1)
    vi_p = ij_p[0]
    vj_p = ij_p[1]

    L, R = _dense_tables(hp, W_proj, b_proj, W_left, b_left, W_right,
                         b_right, n, npad)
    w, d = _pass_a(L, R, vi_p, vj_p, npad, epad)
    o = _pass_b(att_p, d, vi_p, vj_p, w, npad, epad)
    out = _finalize(o, npad, n)
    return out.reshape(1, npad)[:, :n]
